# Initial kernel scaffold; baseline (speedup 1.0000x reference)
import jax
import jax.numpy as jnp
from jax.experimental import pallas as pl


def kernel(x, ln1_w, ln2_w, Wq, Wk, Wv, Wo, qn_w, kn_w, gate_w, gate_bias, Wg, Wu, Wd, Sg, Su, Sd):
    raise NotImplementedError("write your pallas kernel here")


# speedup vs baseline: 1.2541x; 1.2541x over previous
import functools

import jax
import jax.numpy as jnp
from jax import lax
from jax.experimental import pallas as pl
from jax.experimental.pallas import tpu as pltpu

B, L, D = 1, 2048, 1024
H, KVH, DH = 16, 4, 64
E, K, F = 8, 2, 512
EPS = 1e-05
SCALING = 2.5
NSEG = H + 2 * KVH
QKV = NSEG * DH
CDT = jnp.bfloat16
TBLK = 256
NT = L // TBLK

_dn = lambda lc, rc: (((lc,), (rc,)), ((), ()))


def _dot(a, b, lc, rc):
    return lax.dot_general(a, b, _dn(lc, rc), preferred_element_type=jnp.float32)


def _qkv_body(x_ref, w_ref, ln1_ref, scale_ref, out_ref):
    x = x_ref[...]
    ms = jnp.mean(x * x, axis=-1, keepdims=True)
    xn = (x * lax.rsqrt(ms + EPS) * ln1_ref[...]).astype(CDT)
    qkvT = _dot(w_ref[...], xn, 1, 1)
    q3 = qkvT.reshape(NSEG, DH, TBLK)
    ss = jnp.mean(q3 * q3, axis=1, keepdims=True)
    rs = lax.rsqrt(ss + EPS)
    seg = lax.broadcasted_iota(jnp.int32, (NSEG, 1, 1), 0)
    rs = jnp.where(seg >= H + KVH, 1.0, rs)
    out = q3 * rs
    out_ref[...] = (out.reshape(QKV, TBLK) * scale_ref[...]).astype(CDT)


def _qkv_call(x2d, Wqkv, ln1_row, scale_col):
    return pl.pallas_call(
        _qkv_body,
        grid=(NT,),
        in_specs=[
            pl.BlockSpec((TBLK, D), lambda i: (i, 0)),
            pl.BlockSpec((QKV, D), lambda i: (0, 0)),
            pl.BlockSpec((1, D), lambda i: (0, 0)),
            pl.BlockSpec((QKV, 1), lambda i: (0, 0)),
        ],
        out_specs=pl.BlockSpec((QKV, TBLK), lambda i: (0, i)),
        out_shape=jax.ShapeDtypeStruct((QKV, L), CDT),
    )(x2d, Wqkv, ln1_row, scale_col)


BQ = 512
NQ = L // BQ


def _attn_body(q_ref, k_ref, v_ref, out_ref):
    sT = _dot(k_ref[...], q_ref[...], 0, 0) * (DH ** -0.5)
    m = jnp.max(sT, axis=0, keepdims=True)
    p = jnp.exp(sT - m)
    probsT = (p / jnp.sum(p, axis=0, keepdims=True)).astype(CDT)
    ctxT = _dot(v_ref[...], probsT, 1, 0)
    out_ref[...] = ctxT.astype(CDT)


def _attn_call(qkvT):
    return pl.pallas_call(
        _attn_body,
        grid=(H, NQ),
        in_specs=[
            pl.BlockSpec((DH, BQ), lambda h, i: (h, i)),
            pl.BlockSpec((DH, L), lambda h, i: (H + h // (H // KVH), 0)),
            pl.BlockSpec((DH, L), lambda h, i: (H + KVH + h // (H // KVH), 0)),
        ],
        out_specs=pl.BlockSpec((DH, BQ), lambda h, i: (h, i)),
        out_shape=jax.ShapeDtypeStruct((H * DH, L), CDT),
    )(qkvT, qkvT, qkvT)


def _mid_body(xT_ref, ctx_ref, wo_ref, ln2_ref, gw_ref, gb_ref,
              sg_ref, su_ref, sd_ref, tT_ref, base_ref, comb_ref):
    hT = xT_ref[...] + _dot(wo_ref[...], ctx_ref[...], 1, 0)
    ms = jnp.mean(hT * hT, axis=0, keepdims=True)
    tT = hT * lax.rsqrt(ms + EPS) * ln2_ref[...]
    tT_ref[...] = tT.astype(CDT)
    gatesT = _dot(gw_ref[...], tT, 1, 0)
    s = jax.nn.sigmoid(gatesT)
    b = s + gb_ref[...]
    ii = lax.broadcasted_iota(jnp.int32, (E, TBLK), 0)
    m1 = jnp.max(b, axis=0, keepdims=True)
    i1 = jnp.min(jnp.where(b == m1, ii, E), axis=0, keepdims=True)
    o1 = jnp.sum(jnp.where(ii == i1, s, 0.0), axis=0, keepdims=True)
    b2 = jnp.where(ii == i1, -jnp.inf, b)
    m2 = jnp.max(b2, axis=0, keepdims=True)
    i2 = jnp.min(jnp.where(b2 == m2, ii, E), axis=0, keepdims=True)
    o2 = jnp.sum(jnp.where(ii == i2, s, 0.0), axis=0, keepdims=True)
    w = SCALING / (o1 + o2 + 1e-20)
    comb_ref[...] = jnp.where(ii == i1, o1 * w,
                              jnp.where(ii == i2, o2 * w, 0.0))
    tb = tT.astype(CDT)
    gT = _dot(sg_ref[...], tb, 1, 0)
    uT = _dot(su_ref[...], tb, 1, 0)
    actT = (gT * jax.nn.sigmoid(gT) * uT).astype(CDT)
    shT = _dot(sd_ref[...], actT, 1, 0)
    base_ref[...] = hT + shT


def _mid_call(xT, ctxT, Wo, ln2_row, gate_w, gb_col, Sg, Su, Sd):
    return pl.pallas_call(
        _mid_body,
        grid=(NT,),
        in_specs=[
            pl.BlockSpec((D, TBLK), lambda i: (0, i)),
            pl.BlockSpec((H * DH, TBLK), lambda i: (0, i)),
            pl.BlockSpec((D, H * DH), lambda i: (0, 0)),
            pl.BlockSpec((D, 1), lambda i: (0, 0)),
            pl.BlockSpec((E, D), lambda i: (0, 0)),
            pl.BlockSpec((E, 1), lambda i: (0, 0)),
            pl.BlockSpec((F, D), lambda i: (0, 0)),
            pl.BlockSpec((F, D), lambda i: (0, 0)),
            pl.BlockSpec((D, F), lambda i: (0, 0)),
        ],
        out_specs=[
            pl.BlockSpec((D, TBLK), lambda i: (0, i)),
            pl.BlockSpec((D, TBLK), lambda i: (0, i)),
            pl.BlockSpec((E, TBLK), lambda i: (0, i)),
        ],
        out_shape=[
            jax.ShapeDtypeStruct((D, L), CDT),
            jax.ShapeDtypeStruct((D, L), jnp.float32),
            jax.ShapeDtypeStruct((E, L), jnp.float32),
        ],
    )(xT, ctxT, Wo, ln2_row, gate_w, gb_col, Sg, Su, Sd)


def _moe_body(tT_ref, wg_ref, wu_ref, wd_ref, comb_ref, base_ref,
              out_ref, acc_ref):
    e = pl.program_id(0)
    j = pl.program_id(1)
    tb = tT_ref[...]
    gT = _dot(wg_ref[0], tb, 1, 0)
    uT = _dot(wu_ref[0], tb, 1, 0)
    actT = (gT * jax.nn.sigmoid(gT) * uT).astype(CDT)
    yeT = _dot(wd_ref[0], actT, 1, 0)
    wrow = comb_ref[pl.ds(e, 1), :]
    contrib = yeT * wrow
    sl = (slice(None), pl.ds(j * TBLK, TBLK))

    @pl.when(e == 0)
    def _():
        acc_ref[sl] = contrib

    @pl.when(e > 0)
    def _():
        acc_ref[sl] = acc_ref[sl] + contrib

    @pl.when(e == E - 1)
    def _():
        out_ref[...] = acc_ref[sl] + base_ref[...]


def _moe_call(tT, Wg, Wu, Wd, combT, baseT):
    return pl.pallas_call(
        _moe_body,
        grid=(E, NT),
        in_specs=[
            pl.BlockSpec((D, TBLK), lambda e, j: (0, j)),
            pl.BlockSpec((1, F, D), lambda e, j: (e, 0, 0)),
            pl.BlockSpec((1, F, D), lambda e, j: (e, 0, 0)),
            pl.BlockSpec((1, D, F), lambda e, j: (e, 0, 0)),
            pl.BlockSpec((E, TBLK), lambda e, j: (0, j)),
            pl.BlockSpec((D, TBLK), lambda e, j: (0, j)),
        ],
        out_specs=pl.BlockSpec((D, TBLK),
                               lambda e, j: (0, jnp.where(e == E - 1, j, 0))),
        out_shape=jax.ShapeDtypeStruct((D, L), jnp.float32),
        scratch_shapes=[pltpu.VMEM((D, L), jnp.float32)],
    )(tT, Wg, Wu, Wd, combT, baseT)


def kernel(x, ln1_w, ln2_w, Wq, Wk, Wv, Wo, qn_w, kn_w, gate_w, gate_bias,
           Wg, Wu, Wd, Sg, Su, Sd):
    x2d = x.reshape(L, D)
    xT = x2d.T
    Wqkv = jnp.concatenate([Wq, Wk, Wv], axis=0).astype(CDT)
    scale_col = jnp.concatenate(
        [jnp.tile(qn_w, H), jnp.tile(kn_w, KVH), jnp.ones((KVH * DH,), jnp.float32)]
    ).reshape(QKV, 1)
    qkvT = _qkv_call(x2d, Wqkv, ln1_w.reshape(1, D), scale_col)
    ctxT = _attn_call(qkvT)
    tT, baseT, combT = _mid_call(
        xT, ctxT, Wo.astype(CDT), ln2_w.reshape(D, 1), gate_w,
        gate_bias.reshape(E, 1), Sg.astype(CDT), Su.astype(CDT),
        Sd.astype(CDT))
    outT = _moe_call(tT, Wg.astype(CDT), Wu.astype(CDT), Wd.astype(CDT),
                     combT, baseT)
    return outT.T.reshape(B, L, D)


# speedup vs baseline: 1.5427x; 1.2301x over previous
import functools

import jax
import jax.numpy as jnp
from jax import lax
from jax.experimental import pallas as pl
from jax.experimental.pallas import tpu as pltpu

B, L, D = 1, 2048, 1024
H, KVH, DH = 16, 4, 64
E, K, F = 8, 2, 512
EPS = 1e-05
SCALING = 2.5
NSEG = H + 2 * KVH
QKV = NSEG * DH
CDT = jnp.bfloat16
TBLK = 256
NT = L // TBLK

_dn = lambda lc, rc: (((lc,), (rc,)), ((), ()))


def _dot(a, b, lc, rc):
    return lax.dot_general(a, b, _dn(lc, rc), preferred_element_type=jnp.float32)


def _qkv_body(x_ref, w_ref, ln1_ref, scale_ref, out_ref):
    x = x_ref[...]
    ms = jnp.mean(x * x, axis=-1, keepdims=True)
    xn = (x * lax.rsqrt(ms + EPS) * ln1_ref[...]).astype(CDT)
    qkvT = _dot(w_ref[...], xn, 1, 1)
    q3 = qkvT.reshape(NSEG, DH, TBLK)
    ss = jnp.mean(q3 * q3, axis=1, keepdims=True)
    rs = lax.rsqrt(ss + EPS)
    seg = lax.broadcasted_iota(jnp.int32, (NSEG, 1, 1), 0)
    rs = jnp.where(seg >= H + KVH, 1.0, rs)
    out = q3 * rs
    out_ref[...] = (out.reshape(QKV, TBLK) * scale_ref[...]).astype(CDT)


def _qkv_call(x2d, Wqkv, ln1_row, scale_col):
    return pl.pallas_call(
        _qkv_body,
        grid=(NT,),
        in_specs=[
            pl.BlockSpec((TBLK, D), lambda i: (i, 0)),
            pl.BlockSpec((QKV, D), lambda i: (0, 0)),
            pl.BlockSpec((1, D), lambda i: (0, 0)),
            pl.BlockSpec((QKV, 1), lambda i: (0, 0)),
        ],
        out_specs=pl.BlockSpec((QKV, TBLK), lambda i: (0, i)),
        out_shape=jax.ShapeDtypeStruct((QKV, L), CDT),
    )(x2d, Wqkv, ln1_row, scale_col)


BQ = 512
NQ = L // BQ


def _attn_body(q_ref, k_ref, v_ref, out_ref):
    sT = _dot(k_ref[...], q_ref[...], 0, 0)
    p = jnp.exp(sT)
    probsT = (p * (1.0 / jnp.sum(p, axis=0, keepdims=True))).astype(CDT)
    ctxT = _dot(v_ref[...], probsT, 1, 0)
    out_ref[...] = ctxT.astype(CDT)


def _attn_call(qkvT):
    return pl.pallas_call(
        _attn_body,
        grid=(H, NQ),
        in_specs=[
            pl.BlockSpec((DH, BQ), lambda h, i: (h, i)),
            pl.BlockSpec((DH, L), lambda h, i: (H + h // (H // KVH), 0)),
            pl.BlockSpec((DH, L), lambda h, i: (H + KVH + h // (H // KVH), 0)),
        ],
        out_specs=pl.BlockSpec((DH, BQ), lambda h, i: (h, i)),
        out_shape=jax.ShapeDtypeStruct((H * DH, L), CDT),
    )(qkvT, qkvT, qkvT)


def _mid_body(x_ref, ctx_ref, wo_ref, ln2_ref, gw_ref, gb_ref,
              sg_ref, su_ref, sd_ref, tT_ref, base_ref, comb_ref):
    hT = x_ref[...].T + _dot(wo_ref[...], ctx_ref[...], 1, 0)
    ms = jnp.mean(hT * hT, axis=0, keepdims=True)
    tT = hT * lax.rsqrt(ms + EPS) * ln2_ref[...]
    tT_ref[...] = tT.astype(CDT)
    gatesT = _dot(gw_ref[...], tT, 1, 0)
    s = jax.nn.sigmoid(gatesT)
    b = s + gb_ref[...]
    ii = lax.broadcasted_iota(jnp.int32, (E, TBLK), 0)
    m1 = jnp.max(b, axis=0, keepdims=True)
    i1 = jnp.min(jnp.where(b == m1, ii, E), axis=0, keepdims=True)
    o1 = jnp.sum(jnp.where(ii == i1, s, 0.0), axis=0, keepdims=True)
    b2 = jnp.where(ii == i1, -jnp.inf, b)
    m2 = jnp.max(b2, axis=0, keepdims=True)
    i2 = jnp.min(jnp.where(b2 == m2, ii, E), axis=0, keepdims=True)
    o2 = jnp.sum(jnp.where(ii == i2, s, 0.0), axis=0, keepdims=True)
    w = SCALING / (o1 + o2 + 1e-20)
    comb_ref[...] = jnp.where(ii == i1, o1 * w,
                              jnp.where(ii == i2, o2 * w, 0.0))
    tb = tT.astype(CDT)
    gT = _dot(sg_ref[...], tb, 1, 0)
    uT = _dot(su_ref[...], tb, 1, 0)
    actT = (gT * jax.nn.sigmoid(gT) * uT).astype(CDT)
    shT = _dot(sd_ref[...], actT, 1, 0)
    base_ref[...] = hT + shT


def _mid_call(x2d, ctxT, Wo, ln2_row, gate_w, gb_col, Sg, Su, Sd):
    return pl.pallas_call(
        _mid_body,
        grid=(NT,),
        in_specs=[
            pl.BlockSpec((TBLK, D), lambda i: (i, 0)),
            pl.BlockSpec((H * DH, TBLK), lambda i: (0, i)),
            pl.BlockSpec((D, H * DH), lambda i: (0, 0)),
            pl.BlockSpec((D, 1), lambda i: (0, 0)),
            pl.BlockSpec((E, D), lambda i: (0, 0)),
            pl.BlockSpec((E, 1), lambda i: (0, 0)),
            pl.BlockSpec((F, D), lambda i: (0, 0)),
            pl.BlockSpec((F, D), lambda i: (0, 0)),
            pl.BlockSpec((D, F), lambda i: (0, 0)),
        ],
        out_specs=[
            pl.BlockSpec((D, TBLK), lambda i: (0, i)),
            pl.BlockSpec((D, TBLK), lambda i: (0, i)),
            pl.BlockSpec((E, TBLK), lambda i: (0, i)),
        ],
        out_shape=[
            jax.ShapeDtypeStruct((D, L), CDT),
            jax.ShapeDtypeStruct((D, L), jnp.float32),
            jax.ShapeDtypeStruct((E, L), jnp.float32),
        ],
    )(x2d, ctxT, Wo, ln2_row, gate_w, gb_col, Sg, Su, Sd)


def _moe_body(tT_ref, wg_ref, wu_ref, wd_ref, comb_ref, base_ref,
              out_ref, acc_ref):
    e = pl.program_id(0)
    j = pl.program_id(1)
    tb = tT_ref[...]
    gT = _dot(wg_ref[0], tb, 1, 0)
    uT = _dot(wu_ref[0], tb, 1, 0)
    actT = (gT * jax.nn.sigmoid(gT) * uT).astype(CDT)
    yeT = _dot(wd_ref[0], actT, 1, 0)
    wrow = comb_ref[pl.ds(e, 1), :]
    contrib = yeT * wrow
    sl = (slice(None), pl.ds(j * TBLK, TBLK))

    @pl.when(e == 0)
    def _():
        acc_ref[sl] = contrib

    @pl.when(e > 0)
    def _():
        acc_ref[sl] = acc_ref[sl] + contrib

    @pl.when(e == E - 1)
    def _():
        out_ref[...] = (acc_ref[sl] + base_ref[...]).T


def _moe_call(tT, Wg, Wu, Wd, combT, baseT):
    return pl.pallas_call(
        _moe_body,
        grid=(E, NT),
        in_specs=[
            pl.BlockSpec((D, TBLK), lambda e, j: (0, j)),
            pl.BlockSpec((1, F, D), lambda e, j: (e, 0, 0)),
            pl.BlockSpec((1, F, D), lambda e, j: (e, 0, 0)),
            pl.BlockSpec((1, D, F), lambda e, j: (e, 0, 0)),
            pl.BlockSpec((E, TBLK), lambda e, j: (0, j)),
            pl.BlockSpec((D, TBLK), lambda e, j: (0, j)),
        ],
        out_specs=pl.BlockSpec((TBLK, D),
                               lambda e, j: (jnp.where(e == E - 1, j, 0), 0)),
        out_shape=jax.ShapeDtypeStruct((L, D), jnp.float32),
        scratch_shapes=[pltpu.VMEM((D, L), jnp.float32)],
    )(tT, Wg, Wu, Wd, combT, baseT)


def kernel(x, ln1_w, ln2_w, Wq, Wk, Wv, Wo, qn_w, kn_w, gate_w, gate_bias,
           Wg, Wu, Wd, Sg, Su, Sd):
    x2d = x.reshape(L, D)
    Wqkv = jnp.concatenate([Wq, Wk, Wv], axis=0).astype(CDT)
    scale_col = jnp.concatenate(
        [jnp.tile(qn_w, H) * (DH ** -0.5), jnp.tile(kn_w, KVH),
         jnp.ones((KVH * DH,), jnp.float32)]
    ).reshape(QKV, 1)
    qkvT = _qkv_call(x2d, Wqkv, ln1_w.reshape(1, D), scale_col)
    ctxT = _attn_call(qkvT)
    tT, baseT, combT = _mid_call(
        x2d, ctxT, Wo.astype(CDT), ln2_w.reshape(D, 1), gate_w,
        gate_bias.reshape(E, 1), Sg.astype(CDT), Su.astype(CDT),
        Sd.astype(CDT))
    out = _moe_call(tT, Wg.astype(CDT), Wu.astype(CDT), Wd.astype(CDT),
                    combT, baseT)
    return out.reshape(B, L, D)
